# baseline (speedup 1.0000x reference)
import functools

import jax
import jax.numpy as jnp
from jax import lax
from jax.experimental import pallas as pl
from jax.experimental.pallas import tpu as pltpu
from jax.experimental.pallas import tpu_sc as plsc

N_USERS = 1000000
N_ITEMS = 100000
BATCH = 16384
RANK = 32
LANES = 16
NUM_CORES = 2
NUM_SUBCORES = 16
NUM_WORKERS = NUM_CORES * NUM_SUBCORES
BPW = BATCH // NUM_WORKERS
IDX_CHUNK = 128
NCHUNK = BPW // IDX_CHUNK
ROWS_PER_PACK = 128 // RANK


def _relayout_body(xt_ref, out_ref):
    x = xt_ref[...]
    cw = x.shape[1]
    z = x.T.reshape(cw // ROWS_PER_PACK, ROWS_PER_PACK, RANK)
    for s in range(ROWS_PER_PACK):
        out_ref[:, pl.ds(s * RANK, RANK)] = z[:, s, :]


def _pack_rows(xt, n_rows, cw):
    n = xt.shape[1]
    grid = (n + cw - 1) // cw
    return pl.pallas_call(
        _relayout_body,
        out_shape=jax.ShapeDtypeStruct((n_rows, 128), jnp.float32),
        grid=(grid,),
        in_specs=[pl.BlockSpec((RANK, cw), lambda c: (0, c))],
        out_specs=pl.BlockSpec((cw // ROWS_PER_PACK, 128), lambda c: (c, 0)),
    )(xt)


def _mf_body(u_hbm, i_hbm, up_hbm, vp_hbm, mu_hbm, out_hbm,
             u_idx, v_idx, m_idx, u_rows, v_rows, out_v, mu_v, sem):
    wid = lax.axis_index("s") * NUM_CORES + lax.axis_index("c")
    base = wid * BPW

    pltpu.sync_copy(u_hbm.at[pl.ds(base, BPW)], u_idx)
    pltpu.sync_copy(i_hbm.at[pl.ds(base, BPW)], v_idx)
    pltpu.sync_copy(mu_hbm, mu_v)

    for j in range(NCHUNK):
        for t in range(IDX_CHUNK // LANES):
            sl = pl.ds(t * LANES, LANES)
            fsl = pl.ds(j * IDX_CHUNK + t * LANES, LANES)
            m_idx[j, sl] = jax.lax.shift_right_logical(u_idx[fsl], 2)
            m_idx[NCHUNK + j, sl] = jax.lax.shift_right_logical(
                v_idx[fsl], 2)

    mu_vec = mu_v[...]
    lane_iota = lax.iota(jnp.int32, LANES)

    for h in range(2):
        copies = []
        for jj in range(NCHUNK // 2):
            j = h * (NCHUNK // 2) + jj
            copies.append(pltpu.async_copy(
                up_hbm.at[m_idx.at[j]],
                u_rows.at[pl.ds(jj * IDX_CHUNK, IDX_CHUNK)], sem))
            copies.append(pltpu.async_copy(
                vp_hbm.at[m_idx.at[NCHUNK + j]],
                v_rows.at[pl.ds(jj * IDX_CHUNK, IDX_CHUNK)], sem))
        for c in copies:
            c.wait()

        def g_body(g, carry):
            row = g * LANES + lane_iota
            fsl = pl.ds(h * (BPW // 2) + g * LANES, LANES)
            ubase = jax.lax.shift_left(
                jax.lax.bitwise_and(u_idx[fsl], 3), 5)
            vbase = jax.lax.shift_left(
                jax.lax.bitwise_and(v_idx[fsl], 3), 5)
            acc = mu_vec
            for k in range(RANK):
                uv = plsc.load_gather(u_rows, [row, ubase + k])
                vv = plsc.load_gather(v_rows, [row, vbase + k])
                acc = acc + uv * vv
            out_v[pl.ds((h * (BPW // 2)) + g * LANES, LANES)] = acc
            return carry

        lax.fori_loop(0, (BPW // 2) // LANES, g_body, 0, unroll=2)

    pltpu.sync_copy(out_v, out_hbm.at[pl.ds(base, BPW)])


@jax.jit
def kernel(u, i, U, V, mu):
    up = U.reshape(N_USERS // ROWS_PER_PACK, 128)
    vp = V.reshape(N_ITEMS // ROWS_PER_PACK, 128)
    mu_vec = jnp.full((LANES,), mu, jnp.float32)
    mesh = plsc.VectorSubcoreMesh(
        core_axis_name="c", subcore_axis_name="s",
        num_cores=NUM_CORES, num_subcores=NUM_SUBCORES)
    run = pl.kernel(
        _mf_body,
        out_type=jax.ShapeDtypeStruct((BATCH,), jnp.float32),
        mesh=mesh,
        scratch_types=[
            pltpu.VMEM((BPW,), jnp.int32),
            pltpu.VMEM((BPW,), jnp.int32),
            pltpu.VMEM((2 * NCHUNK, IDX_CHUNK), jnp.int32),
            pltpu.VMEM((BPW // 2, 128), jnp.float32),
            pltpu.VMEM((BPW // 2, 128), jnp.float32),
            pltpu.VMEM((BPW,), jnp.float32),
            pltpu.VMEM((LANES,), jnp.float32),
            pltpu.SemaphoreType.DMA,
        ],
        compiler_params=pltpu.CompilerParams(
            needs_layout_passes=False, use_tc_tiling_on_sc=True),
    )
    return run(u.astype(jnp.int32), i.astype(jnp.int32), up, vp, mu_vec)


# speedup vs baseline: 3.5232x; 3.5232x over previous
import functools

import jax
import jax.numpy as jnp
from jax import lax
from jax.experimental import pallas as pl
from jax.experimental.pallas import tpu as pltpu
from jax.experimental.pallas import tpu_sc as plsc

N_USERS = 1000000
N_ITEMS = 100000
BATCH = 16384
RANK = 32
LANES = 16
NUM_CORES = 2
NUM_SUBCORES = 16
NUM_WORKERS = NUM_CORES * NUM_SUBCORES
BPW = BATCH // NUM_WORKERS
IDX_CHUNK = 128
NCHUNK = BPW // IDX_CHUNK
STRATA = 128 // RANK

U_LOG2S = 18
V_LOG2S = 15
CW4 = 4096


def _relayout_body(x0_ref, x1_ref, x2_ref, x3_ref, out_ref):
    xs = jnp.concatenate(
        [x0_ref[...], x1_ref[...], x2_ref[...], x3_ref[...]], axis=0)
    eye = jnp.eye(STRATA * RANK, dtype=jnp.float32)
    out_ref[...] = lax.dot_general(
        xs, eye, (((0,), (0,)), ((), ())),
        preferred_element_type=jnp.float32)


def _pack_rows(xt, log2s):
    stride_blocks = (1 << log2s) // CW4
    n_rows = 1 << log2s
    grid = n_rows // CW4
    max_block = (xt.shape[1] - 1) // CW4

    def in_spec(s):
        return pl.BlockSpec(
            (RANK, CW4),
            lambda c, s=s: (0, jnp.minimum(s * stride_blocks + c, max_block)))

    return pl.pallas_call(
        _relayout_body,
        out_shape=jax.ShapeDtypeStruct((n_rows, STRATA * RANK), jnp.float32),
        grid=(grid,),
        in_specs=[in_spec(s) for s in range(STRATA)],
        out_specs=pl.BlockSpec((CW4, STRATA * RANK), lambda c: (c, 0)),
    )(xt, xt, xt, xt)


def _mf_body(u_hbm, i_hbm, up_hbm, vp_hbm, mu_hbm, out_hbm,
             u_idx, v_idx, m_idx, u_rows, v_rows, out_v, mu_v, sem):
    wid = lax.axis_index("s") * NUM_CORES + lax.axis_index("c")
    base = wid * BPW

    pltpu.sync_copy(u_hbm.at[pl.ds(base, BPW)], u_idx)
    pltpu.sync_copy(i_hbm.at[pl.ds(base, BPW)], v_idx)
    pltpu.sync_copy(mu_hbm, mu_v)

    for j in range(NCHUNK):
        for t in range(IDX_CHUNK // LANES):
            sl = pl.ds(t * LANES, LANES)
            fsl = pl.ds(j * IDX_CHUNK + t * LANES, LANES)
            m_idx[j, sl] = jax.lax.bitwise_and(u_idx[fsl], (1 << U_LOG2S) - 1)
            m_idx[NCHUNK + j, sl] = jax.lax.bitwise_and(
                v_idx[fsl], (1 << V_LOG2S) - 1)

    mu_vec = mu_v[...]
    lane_iota = lax.iota(jnp.int32, LANES)

    for h in range(2):
        copies = []
        for jj in range(NCHUNK // 2):
            j = h * (NCHUNK // 2) + jj
            copies.append(pltpu.async_copy(
                up_hbm.at[m_idx.at[j]],
                u_rows.at[pl.ds(jj * IDX_CHUNK, IDX_CHUNK)], sem))
            copies.append(pltpu.async_copy(
                vp_hbm.at[m_idx.at[NCHUNK + j]],
                v_rows.at[pl.ds(jj * IDX_CHUNK, IDX_CHUNK)], sem))
        for c in copies:
            c.wait()

        def g_body(g, carry):
            row = g * LANES + lane_iota
            fsl = pl.ds(h * (BPW // 2) + g * LANES, LANES)
            ubase = jax.lax.shift_left(
                jax.lax.shift_right_logical(u_idx[fsl], U_LOG2S), 5)
            vbase = jax.lax.shift_left(
                jax.lax.shift_right_logical(v_idx[fsl], V_LOG2S), 5)
            acc = mu_vec
            for k in range(RANK):
                uv = plsc.load_gather(u_rows, [row, ubase + k])
                vv = plsc.load_gather(v_rows, [row, vbase + k])
                acc = acc + uv * vv
            out_v[pl.ds((h * (BPW // 2)) + g * LANES, LANES)] = acc
            return carry

        lax.fori_loop(0, (BPW // 2) // LANES, g_body, 0, unroll=2)

    pltpu.sync_copy(out_v, out_hbm.at[pl.ds(base, BPW)])


@jax.jit
def kernel(u, i, U, V, mu):
    up = _pack_rows(U.T, U_LOG2S)
    vp = _pack_rows(V.T, V_LOG2S)
    mu_vec = jnp.full((LANES,), mu, jnp.float32)
    mesh = plsc.VectorSubcoreMesh(
        core_axis_name="c", subcore_axis_name="s",
        num_cores=NUM_CORES, num_subcores=NUM_SUBCORES)
    run = pl.kernel(
        _mf_body,
        out_type=jax.ShapeDtypeStruct((BATCH,), jnp.float32),
        mesh=mesh,
        scratch_types=[
            pltpu.VMEM((BPW,), jnp.int32),
            pltpu.VMEM((BPW,), jnp.int32),
            pltpu.VMEM((2 * NCHUNK, IDX_CHUNK), jnp.int32),
            pltpu.VMEM((BPW // 2, 128), jnp.float32),
            pltpu.VMEM((BPW // 2, 128), jnp.float32),
            pltpu.VMEM((BPW,), jnp.float32),
            pltpu.VMEM((LANES,), jnp.float32),
            pltpu.SemaphoreType.DMA,
        ],
        compiler_params=pltpu.CompilerParams(
            needs_layout_passes=False, use_tc_tiling_on_sc=True),
    )
    return run(u.astype(jnp.int32), i.astype(jnp.int32), up, vp, mu_vec)


# speedup vs baseline: 3.9628x; 1.1248x over previous
import functools

import jax
import jax.numpy as jnp
from jax import lax
from jax.experimental import pallas as pl
from jax.experimental.pallas import tpu as pltpu
from jax.experimental.pallas import tpu_sc as plsc

N_USERS = 1000000
N_ITEMS = 100000
BATCH = 16384
RANK = 32
LANES = 16
NUM_CORES = 2
NUM_SUBCORES = 16
NUM_WORKERS = NUM_CORES * NUM_SUBCORES
BPW = BATCH // NUM_WORKERS
IDX_CHUNK = 128
NCHUNK = BPW // IDX_CHUNK
STRATA = 128 // RANK

U_LOG2S = 18
V_LOG2S = 15
CW4 = 8192


def _relayout_body(x0_ref, x1_ref, x2_ref, x3_ref, out_ref):
    xs = jnp.concatenate(
        [x0_ref[...], x1_ref[...], x2_ref[...], x3_ref[...]], axis=0)
    eye = jnp.eye(STRATA * RANK, dtype=jnp.float32)
    out_ref[...] = lax.dot_general(
        xs, eye, (((0,), (0,)), ((), ())),
        preferred_element_type=jnp.float32)


def _pack_rows(xt, log2s):
    stride_blocks = (1 << log2s) // CW4
    n_rows = 1 << log2s
    grid = n_rows // CW4
    max_block = (xt.shape[1] - 1) // CW4

    def in_spec(s):
        return pl.BlockSpec(
            (RANK, CW4),
            lambda c, s=s: (0, jnp.minimum(s * stride_blocks + c, max_block)))

    return pl.pallas_call(
        _relayout_body,
        out_shape=jax.ShapeDtypeStruct((n_rows, STRATA * RANK), jnp.float32),
        grid=(grid,),
        in_specs=[in_spec(s) for s in range(STRATA)],
        out_specs=pl.BlockSpec((CW4, STRATA * RANK), lambda c: (c, 0)),
    )(xt, xt, xt, xt)


def _mf_body(u_hbm, i_hbm, up_hbm, vp_hbm, mu_hbm, out_hbm,
             u_idx, v_idx, m_idx, u_rows, v_rows, out_v, mu_v, sem):
    wid = lax.axis_index("s") * NUM_CORES + lax.axis_index("c")
    base = wid * BPW

    pltpu.sync_copy(u_hbm.at[pl.ds(base, BPW)], u_idx)
    pltpu.sync_copy(i_hbm.at[pl.ds(base, BPW)], v_idx)
    pltpu.sync_copy(mu_hbm, mu_v)

    for j in range(NCHUNK):
        for t in range(IDX_CHUNK // LANES):
            sl = pl.ds(t * LANES, LANES)
            fsl = pl.ds(j * IDX_CHUNK + t * LANES, LANES)
            m_idx[j, sl] = jax.lax.bitwise_and(u_idx[fsl], (1 << U_LOG2S) - 1)
            m_idx[NCHUNK + j, sl] = jax.lax.bitwise_and(
                v_idx[fsl], (1 << V_LOG2S) - 1)

    mu_vec = mu_v[...]
    lane_iota = lax.iota(jnp.int32, LANES)

    for h in range(2):
        copies = []
        for jj in range(NCHUNK // 2):
            j = h * (NCHUNK // 2) + jj
            copies.append(pltpu.async_copy(
                up_hbm.at[m_idx.at[j]],
                u_rows.at[pl.ds(jj * IDX_CHUNK, IDX_CHUNK)], sem))
            copies.append(pltpu.async_copy(
                vp_hbm.at[m_idx.at[NCHUNK + j]],
                v_rows.at[pl.ds(jj * IDX_CHUNK, IDX_CHUNK)], sem))
        for c in copies:
            c.wait()

        def g_body(g, carry):
            row = g * LANES + lane_iota
            fsl = pl.ds(h * (BPW // 2) + g * LANES, LANES)
            ubase = jax.lax.shift_left(
                jax.lax.shift_right_logical(u_idx[fsl], U_LOG2S), 5)
            vbase = jax.lax.shift_left(
                jax.lax.shift_right_logical(v_idx[fsl], V_LOG2S), 5)
            acc = mu_vec
            for k in range(RANK):
                uv = plsc.load_gather(u_rows, [row, ubase + k])
                vv = plsc.load_gather(v_rows, [row, vbase + k])
                acc = acc + uv * vv
            out_v[pl.ds((h * (BPW // 2)) + g * LANES, LANES)] = acc
            return carry

        lax.fori_loop(0, (BPW // 2) // LANES, g_body, 0, unroll=2)

    pltpu.sync_copy(out_v, out_hbm.at[pl.ds(base, BPW)])


@jax.jit
def kernel(u, i, U, V, mu):
    up = _pack_rows(U.T, U_LOG2S)
    vp = _pack_rows(V.T, V_LOG2S)
    mu_vec = jnp.full((LANES,), mu, jnp.float32)
    mesh = plsc.VectorSubcoreMesh(
        core_axis_name="c", subcore_axis_name="s",
        num_cores=NUM_CORES, num_subcores=NUM_SUBCORES)
    run = pl.kernel(
        _mf_body,
        out_type=jax.ShapeDtypeStruct((BATCH,), jnp.float32),
        mesh=mesh,
        scratch_types=[
            pltpu.VMEM((BPW,), jnp.int32),
            pltpu.VMEM((BPW,), jnp.int32),
            pltpu.VMEM((2 * NCHUNK, IDX_CHUNK), jnp.int32),
            pltpu.VMEM((BPW // 2, 128), jnp.float32),
            pltpu.VMEM((BPW // 2, 128), jnp.float32),
            pltpu.VMEM((BPW,), jnp.float32),
            pltpu.VMEM((LANES,), jnp.float32),
            pltpu.SemaphoreType.DMA,
        ],
        compiler_params=pltpu.CompilerParams(
            needs_layout_passes=False, use_tc_tiling_on_sc=True),
    )
    return run(u.astype(jnp.int32), i.astype(jnp.int32), up, vp, mu_vec)


# speedup vs baseline: 4.8620x; 1.2269x over previous
import functools

import jax
import jax.numpy as jnp
from jax import lax
from jax.experimental import pallas as pl
from jax.experimental.pallas import tpu as pltpu
from jax.experimental.pallas import tpu_sc as plsc

N_USERS = 1000000
N_ITEMS = 100000
BATCH = 16384
RANK = 32
LANES = 16
NUM_CORES = 2
NUM_SUBCORES = 16
NUM_WORKERS = NUM_CORES * NUM_SUBCORES
BPW = BATCH // NUM_WORKERS
IDX_CHUNK = 128
NCHUNK = BPW // IDX_CHUNK
STRATA = 8

U_LOG2S = 17
V_LOG2S = 14
CW4 = 8192


def _relayout_body(*refs):
    x = [r[...] for r in refs[:STRATA]]
    out_ref = refs[STRATA]
    eye = jnp.eye(128, dtype=jnp.float32)
    dims = (((0,), (0,)), ((), ()))
    ya = lax.dot_general(jnp.concatenate(x[0::2], axis=0), eye, dims,
                         preferred_element_type=jnp.float32)
    yb = lax.dot_general(jnp.concatenate(x[1::2], axis=0), eye, dims,
                         preferred_element_type=jnp.float32)
    za = lax.bitcast_convert_type(
        ya.astype(jnp.bfloat16), jnp.uint16).astype(jnp.uint32)
    zb = lax.bitcast_convert_type(
        yb.astype(jnp.bfloat16), jnp.uint16).astype(jnp.uint32)
    out_ref[...] = lax.bitcast_convert_type(
        za | (zb << 16), jnp.float32)


def _pack_rows(xt, log2s):
    stride_blocks = (1 << log2s) // CW4
    n_rows = 1 << log2s
    grid = n_rows // CW4
    max_block = (xt.shape[1] - 1) // CW4

    def in_spec(s):
        return pl.BlockSpec(
            (RANK, CW4),
            lambda c, s=s: (0, jnp.minimum(s * stride_blocks + c, max_block)))

    return pl.pallas_call(
        _relayout_body,
        out_shape=jax.ShapeDtypeStruct((n_rows, 128), jnp.float32),
        grid=(grid,),
        in_specs=[in_spec(s) for s in range(STRATA)],
        out_specs=pl.BlockSpec((CW4, 128), lambda c: (c, 0)),
    )(*([xt] * STRATA))


def _mf_body(u_hbm, i_hbm, up_hbm, vp_hbm, mu_hbm, out_hbm,
             u_idx, v_idx, m_idx, u_rows, v_rows, out_v, mu_v, sem):
    wid = lax.axis_index("s") * NUM_CORES + lax.axis_index("c")
    base = wid * BPW

    pltpu.sync_copy(u_hbm.at[pl.ds(base, BPW)], u_idx)
    pltpu.sync_copy(i_hbm.at[pl.ds(base, BPW)], v_idx)
    pltpu.sync_copy(mu_hbm, mu_v)

    for j in range(NCHUNK):
        for t in range(IDX_CHUNK // LANES):
            sl = pl.ds(t * LANES, LANES)
            fsl = pl.ds(j * IDX_CHUNK + t * LANES, LANES)
            m_idx[j, sl] = jax.lax.bitwise_and(u_idx[fsl], (1 << U_LOG2S) - 1)
            m_idx[NCHUNK + j, sl] = jax.lax.bitwise_and(
                v_idx[fsl], (1 << V_LOG2S) - 1)

    mu_vec = mu_v[...]
    lane_iota = lax.iota(jnp.int32, LANES)
    one = jnp.ones((LANES,), jnp.int32)

    def pick(rows_ref, row, colbase, parity, k):
        word = plsc.load_gather(rows_ref, [row, colbase + k])
        lo, hi = plsc.unpack(plsc.bitcast(word, jnp.bfloat16),
                             format=plsc.PackFormat.INTERLEAVED)
        return jnp.where(parity == 0, lo, hi)

    for h in range(2):
        copies = []
        for jj in range(NCHUNK // 2):
            j = h * (NCHUNK // 2) + jj
            copies.append(pltpu.async_copy(
                up_hbm.at[m_idx.at[j]],
                u_rows.at[pl.ds(jj * IDX_CHUNK, IDX_CHUNK)], sem))
            copies.append(pltpu.async_copy(
                vp_hbm.at[m_idx.at[NCHUNK + j]],
                v_rows.at[pl.ds(jj * IDX_CHUNK, IDX_CHUNK)], sem))
        for c in copies:
            c.wait()

        def g_body(g, carry):
            row = g * LANES + lane_iota
            fsl = pl.ds(h * (BPW // 2) + g * LANES, LANES)
            uvec = u_idx[fsl]
            vvec = v_idx[fsl]
            ubase = jax.lax.shift_left(
                jax.lax.shift_right_logical(uvec, U_LOG2S + 1), 5)
            vbase = jax.lax.shift_left(
                jax.lax.shift_right_logical(vvec, V_LOG2S + 1), 5)
            upar = jax.lax.bitwise_and(
                jax.lax.shift_right_logical(uvec, U_LOG2S), one)
            vpar = jax.lax.bitwise_and(
                jax.lax.shift_right_logical(vvec, V_LOG2S), one)
            acc = mu_vec
            for k in range(RANK):
                uv = pick(u_rows, row, ubase, upar, k)
                vv = pick(v_rows, row, vbase, vpar, k)
                acc = acc + uv * vv
            out_v[pl.ds((h * (BPW // 2)) + g * LANES, LANES)] = acc
            return carry

        lax.fori_loop(0, (BPW // 2) // LANES, g_body, 0, unroll=2)

    pltpu.sync_copy(out_v, out_hbm.at[pl.ds(base, BPW)])


@jax.jit
def kernel(u, i, U, V, mu):
    up = _pack_rows(U.T, U_LOG2S)
    vp = _pack_rows(V.T, V_LOG2S)
    mu_vec = jnp.full((LANES,), mu, jnp.float32)
    mesh = plsc.VectorSubcoreMesh(
        core_axis_name="c", subcore_axis_name="s",
        num_cores=NUM_CORES, num_subcores=NUM_SUBCORES)
    run = pl.kernel(
        _mf_body,
        out_type=jax.ShapeDtypeStruct((BATCH,), jnp.float32),
        mesh=mesh,
        scratch_types=[
            pltpu.VMEM((BPW,), jnp.int32),
            pltpu.VMEM((BPW,), jnp.int32),
            pltpu.VMEM((2 * NCHUNK, IDX_CHUNK), jnp.int32),
            pltpu.VMEM((BPW // 2, 128), jnp.float32),
            pltpu.VMEM((BPW // 2, 128), jnp.float32),
            pltpu.VMEM((BPW,), jnp.float32),
            pltpu.VMEM((LANES,), jnp.float32),
            pltpu.SemaphoreType.DMA,
        ],
        compiler_params=pltpu.CompilerParams(
            needs_layout_passes=False, use_tc_tiling_on_sc=True),
    )
    return run(u.astype(jnp.int32), i.astype(jnp.int32), up, vp, mu_vec)
